# Initial kernel scaffold; baseline (speedup 1.0000x reference)
import jax
import jax.numpy as jnp
from jax.experimental import pallas as pl


def kernel(x, edge_index, W1, b1, W_mu, b_mu, W_lv, b_lv):
    raise NotImplementedError("write your pallas kernel here")


# speedup vs baseline: 15.5951x; 15.5951x over previous
import functools

import jax
import jax.numpy as jnp
from jax import lax
from jax.experimental import pallas as pl
from jax.experimental.pallas import tpu as pltpu
from jax.experimental.pallas import tpu_sc as plsc

NC = 2
NS = 16
NW = NC * NS
WIN = 128
PAD_ROWS = 112
_HIGHEST = jax.lax.Precision.HIGHEST


def _flat_wid():
    return lax.axis_index("c") * NS + lax.axis_index("s")


def _deg_partials(dst3, ones_rows, zeros_d, n_acc, width):
    k_win = dst3.shape[1]
    rows_sub = n_acc // NS
    mesh = plsc.VectorSubcoreMesh(core_axis_name="c", subcore_axis_name="s")

    @functools.partial(
        pl.kernel,
        mesh=mesh,
        out_type=jax.ShapeDtypeStruct((NC, n_acc, width), jnp.float32),
        scratch_types=[
            pltpu.VMEM((k_win, WIN), jnp.int32),
            pltpu.VMEM((WIN, width), jnp.float32),
            pltpu.VMEM_SHARED((n_acc, width), jnp.float32),
        ],
    )
    def deg_kernel(dst_hbm, ones_hbm, zeros_hbm, out_hbm, dst_v, ones_v, acc_sh):
        c = lax.axis_index("c")
        s = lax.axis_index("s")
        wid = _flat_wid()
        pltpu.sync_copy(zeros_hbm.at[pl.ds(s * rows_sub, rows_sub)],
                        acc_sh.at[pl.ds(s * rows_sub, rows_sub)])
        pltpu.sync_copy(dst_hbm.at[wid], dst_v)
        pltpu.sync_copy(ones_hbm, ones_v)
        plsc.subcore_barrier()

        @pl.loop(0, k_win)
        def _(j):
            pltpu.sync_copy(ones_v, acc_sh.at[dst_v.at[j]], add=True)

        plsc.subcore_barrier()
        pltpu.sync_copy(acc_sh.at[pl.ds(s * rows_sub, rows_sub)],
                        out_hbm.at[c, pl.ds(s * rows_sub, rows_sub)])

    return deg_kernel(dst3, ones_rows, zeros_d)


def _agg_partials(src3, dst3, p, zeros_d, n, n_acc):
    del n
    k_win = src3.shape[1]
    d_feat = p.shape[1]
    rows_sub = n_acc // NS
    mesh = plsc.VectorSubcoreMesh(core_axis_name="c", subcore_axis_name="s")

    @functools.partial(
        pl.kernel,
        mesh=mesh,
        out_type=jax.ShapeDtypeStruct((NC, n_acc, d_feat), jnp.float32),
        scratch_types=[
            pltpu.VMEM((k_win, WIN), jnp.int32),
            pltpu.VMEM((k_win, WIN), jnp.int32),
            pltpu.VMEM((WIN, d_feat), jnp.float32),
            pltpu.VMEM_SHARED((n_acc, d_feat), jnp.float32),
            pltpu.SemaphoreType.DMA,
        ],
    )
    def agg_kernel(src_hbm, dst_hbm, p_hbm, zeros_hbm, out_hbm,
                   src_v, dst_v, rows_v, acc_sh, sem):
        c = lax.axis_index("c")
        s = lax.axis_index("s")
        wid = _flat_wid()
        pltpu.sync_copy(zeros_hbm.at[pl.ds(s * rows_sub, rows_sub)],
                        acc_sh.at[pl.ds(s * rows_sub, rows_sub)])
        pltpu.sync_copy(src_hbm.at[wid], src_v)
        pltpu.sync_copy(dst_hbm.at[wid], dst_v)
        plsc.subcore_barrier()

        @pl.loop(0, k_win)
        def _(j):
            pltpu.async_copy(p_hbm.at[src_v.at[j]], rows_v, sem).wait()
            pltpu.sync_copy(rows_v, acc_sh.at[dst_v.at[j]], add=True)

        plsc.subcore_barrier()
        pltpu.sync_copy(acc_sh.at[pl.ds(s * rows_sub, rows_sub)],
                        out_hbm.at[c, pl.ds(s * rows_sub, rows_sub)])

    return agg_kernel(src3, dst3, p, zeros_d)


def _dinv_from_parts(degp, n):
    deg = degp[0, :n, 0:1] + degp[1, :n, 0:1] + 1.0
    return 1.0 / jnp.sqrt(deg)


def _mm_body(x_ref, w_ref, o_ref):
    o_ref[...] = jnp.dot(x_ref[...], w_ref[...],
                         preferred_element_type=jnp.float32,
                         precision=_HIGHEST)


def _scale_body(h_ref, degp_ref, p_ref):
    n = h_ref.shape[0]
    p_ref[...] = h_ref[...] * _dinv_from_parts(degp_ref[...], n)


def _hidden_body(a_ref, p1_ref, degp_ref, b1_ref, p2_ref):
    n = p1_ref.shape[0]
    dinv = _dinv_from_parts(degp_ref[...], n)
    a = a_ref[...]
    pre = (a[0, :n] + a[1, :n] + p1_ref[...]) * dinv + b1_ref[...]
    p2_ref[...] = jnp.maximum(pre, 0.0) * dinv


def _final_body(a_ref, p2_ref, degp_ref, wmu_ref, bmu_ref, wlv_ref, blv_ref,
                mu_ref, lv_ref):
    n = p2_ref.shape[0]
    dinv = _dinv_from_parts(degp_ref[...], n)
    a = a_ref[...]
    z = (a[0, :n] + a[1, :n] + p2_ref[...]) * dinv
    mu_ref[...] = jnp.dot(z, wmu_ref[...], preferred_element_type=jnp.float32,
                          precision=_HIGHEST) + bmu_ref[...]
    lv_ref[...] = jnp.dot(z, wlv_ref[...], preferred_element_type=jnp.float32,
                          precision=_HIGHEST) + blv_ref[...]


def _f32(*shape):
    return jax.ShapeDtypeStruct(shape, jnp.float32)


def kernel(x, edge_index, W1, b1, W_mu, b_mu, W_lv, b_lv):
    n, d_in = x.shape
    h1_dim = W1.shape[1]
    h2_dim = W_mu.shape[1]
    e = edge_index.shape[1]

    chunk = NW * WIN
    e_pad = -(-e // chunk) * chunk
    pad = e_pad - e
    k_win = e_pad // chunk
    n_acc = n + PAD_ROWS

    src = edge_index[0]
    dst = edge_index[1]
    pad_src = jnp.zeros((pad,), jnp.int32)
    pad_dst = n + (jnp.arange(pad, dtype=jnp.int32) % PAD_ROWS)
    src3 = jnp.concatenate([src, pad_src]).reshape(NW, k_win, WIN)
    dst3 = jnp.concatenate([dst, pad_dst]).reshape(NW, k_win, WIN)

    ones_rows = jnp.ones((WIN, h1_dim), jnp.float32)
    zeros_d = jnp.zeros((n_acc, h1_dim), jnp.float32)

    degp = _deg_partials(dst3, ones_rows, zeros_d, n_acc, h1_dim)
    h1 = pl.pallas_call(_mm_body, out_shape=_f32(n, h1_dim))(x, W1)

    p1 = pl.pallas_call(_scale_body, out_shape=_f32(n, h1_dim))(h1, degp)

    a1 = _agg_partials(src3, dst3, p1, zeros_d, n, n_acc)

    p2 = pl.pallas_call(_hidden_body, out_shape=_f32(n, h1_dim))(
        a1, p1, degp, b1.reshape(1, h1_dim))

    a2 = _agg_partials(src3, dst3, p2, zeros_d, n, n_acc)

    mu, lv = pl.pallas_call(
        _final_body, out_shape=(_f32(n, h2_dim), _f32(n, h2_dim)))(
        a2, p2, degp, W_mu, b_mu.reshape(1, h2_dim), W_lv,
        b_lv.reshape(1, h2_dim))
    return (mu, lv)
